# Initial kernel scaffold; baseline (speedup 1.0000x reference)
import jax
import jax.numpy as jnp
from jax.experimental import pallas as pl


def kernel(x, batch_id, W, b):
    raise NotImplementedError("write your pallas kernel here")


# speedup vs baseline: 3.8924x; 3.8924x over previous
import functools

import jax
import jax.numpy as jnp
from jax import lax
from jax.experimental import pallas as pl
from jax.experimental.pallas import tpu as pltpu

N = 100000
D = 128
OUT = 128
S = 512
R = 2000
NB = N // R


def _body(x_ref, ids_ref, w1t_ref, w2t_ref, b_ref, out_ref,
          sums_ref, counts_ref, g_ref):
    phase = pl.program_id(0)
    i = pl.program_id(1)

    ids = ids_ref[...]
    onehot = (ids == lax.broadcasted_iota(jnp.int32, (R, S), 1)
              ).astype(jnp.bfloat16)

    @pl.when(jnp.logical_and(phase == 0, i == 0))
    def _init():
        sums_ref[...] = jnp.zeros_like(sums_ref)
        counts_ref[...] = jnp.zeros_like(counts_ref)

    @pl.when(phase == 0)
    def _accum():
        xb = x_ref[...].astype(jnp.bfloat16)
        part = lax.dot_general(onehot, xb, (((0,), (0,)), ((), ())),
                               preferred_element_type=jnp.float32)
        sums_ref[...] += part
        counts_ref[...] += jnp.sum(onehot.astype(jnp.float32), axis=0,
                                   keepdims=True)

    @pl.when(jnp.logical_and(phase == 1, i == 0))
    def _make_g():
        counts = jnp.maximum(counts_ref[...], 1.0)
        mean = sums_ref[...] * (1.0 / counts).T
        g = jnp.dot(mean, w2t_ref[...],
                    preferred_element_type=jnp.float32) + b_ref[...]
        g_ref[...] = g.astype(jnp.bfloat16)

    @pl.when(phase == 1)
    def _emit():
        main = jnp.dot(x_ref[...], w1t_ref[...],
                       preferred_element_type=jnp.float32)
        gath = jnp.dot(onehot, g_ref[...],
                       preferred_element_type=jnp.float32)
        out_ref[...] = main + gath


@jax.jit
def kernel(x, batch_id, W, b):
    ids = batch_id.astype(jnp.int32).reshape(N, 1)
    wt = W.T
    w1t = wt[:D]
    w2t = wt[D:]
    brow = b.reshape(1, OUT)

    grid = (2, NB)
    return pl.pallas_call(
        _body,
        grid=grid,
        in_specs=[
            pl.BlockSpec((R, D), lambda p, i: (i, 0)),
            pl.BlockSpec((R, 1), lambda p, i: (i, 0)),
            pl.BlockSpec((D, OUT), lambda p, i: (0, 0)),
            pl.BlockSpec((D, OUT), lambda p, i: (0, 0)),
            pl.BlockSpec((1, OUT), lambda p, i: (0, 0)),
        ],
        out_specs=pl.BlockSpec((R, OUT), lambda p, i: (jnp.where(p == 1, i, 0), 0)),
        out_shape=jax.ShapeDtypeStruct((N, OUT), jnp.float32),
        scratch_shapes=[
            pltpu.VMEM((S, D), jnp.float32),
            pltpu.VMEM((1, S), jnp.float32),
            pltpu.VMEM((S, OUT), jnp.bfloat16),
        ],
    )(x, ids, w1t, w2t, brow)


# speedup vs baseline: 4.0211x; 1.0331x over previous
import functools

import jax
import jax.numpy as jnp
from jax import lax
from jax.experimental import pallas as pl
from jax.experimental.pallas import tpu as pltpu
from jax.experimental.pallas import tpu_sc as plsc

N = 100000
D = 128
OUT = 128
S = 512
R = 2000
NB = N // R

NC = 2
NS = 16
NW = NC * NS
CH = 128
NFULL = N // CH
TAIL = N - NFULL * CH
ITERS = (NFULL + NW - 1) // NW
CW = 128


def _make_sc_segsum(cw):
    def body(x_hbm, ids_hbm, zsum_hbm, zcnt_hbm, ones_hbm,
             sums_hbm, cnts_hbm,
             xbuf, idxbuf, onesbuf, xtail, idxtail, acc, cacc):
        cid = lax.axis_index("c")
        sid = lax.axis_index("s")
        wid = cid * NS + sid

        @pl.when(sid == 0)
        def _init():
            pltpu.sync_copy(zsum_hbm, acc)
            pltpu.sync_copy(zcnt_hbm, cacc)

        pltpu.sync_copy(ones_hbm, onesbuf)
        plsc.subcore_barrier()

        def step(it, carry):
            c = wid + it * NW

            @pl.when(c < NFULL)
            def _chunk():
                off = c * CH
                pltpu.sync_copy(x_hbm.at[pl.ds(off, CH)], xbuf)
                pltpu.sync_copy(ids_hbm.at[pl.ds(off, CH)], idxbuf)
                pltpu.sync_copy(xbuf, acc.at[idxbuf], add=True)
                pltpu.sync_copy(onesbuf, cacc.at[idxbuf], add=True)

            return carry

        lax.fori_loop(0, ITERS, step, 0)

        @pl.when(wid == 0)
        def _tail():
            pltpu.sync_copy(x_hbm.at[pl.ds(NFULL * CH, TAIL)], xtail)
            pltpu.sync_copy(ids_hbm.at[pl.ds(NFULL * CH, TAIL)], idxtail)
            pltpu.sync_copy(xtail, acc.at[idxtail], add=True)
            pltpu.sync_copy(onesbuf.at[pl.ds(0, TAIL)], cacc.at[idxtail],
                            add=True)

        plsc.subcore_barrier()

        @pl.when(sid == 0)
        def _out():
            pltpu.sync_copy(acc, sums_hbm.at[cid])
            pltpu.sync_copy(cacc, cnts_hbm.at[cid])

    return functools.partial(
        pl.kernel,
        mesh=plsc.VectorSubcoreMesh(core_axis_name="c", subcore_axis_name="s"),
        out_type=(jax.ShapeDtypeStruct((NC, S, D), jnp.float32),
                  jax.ShapeDtypeStruct((NC, S, cw), jnp.float32)),
        scratch_types=[
            pltpu.VMEM((CH, D), jnp.float32),
            pltpu.VMEM((CH,), jnp.int32),
            pltpu.VMEM((CH, cw), jnp.float32),
            pltpu.VMEM((TAIL, D), jnp.float32),
            pltpu.VMEM((TAIL,), jnp.int32),
            pltpu.VMEM_SHARED((S, D), jnp.float32),
            pltpu.VMEM_SHARED((S, cw), jnp.float32),
        ],
    )(body)


_sc_segsum = _make_sc_segsum(CW)


def _tc_body(x_ref, ids_ref, w1t_ref, w2t_ref, b_ref, sums_ref, cnts_ref,
             out_ref, g_ref):
    i = pl.program_id(0)

    @pl.when(i == 0)
    def _make_g():
        ssum = sums_ref[0] + sums_ref[1]
        counts = cnts_ref[0, :, 0:1] + cnts_ref[1, :, 0:1]
        mean = ssum * (1.0 / jnp.maximum(counts, 1.0))
        g = jnp.dot(mean, w2t_ref[...],
                    preferred_element_type=jnp.float32) + b_ref[...]
        g_ref[...] = g.astype(jnp.bfloat16)

    ids = ids_ref[...]
    onehot = (ids == lax.broadcasted_iota(jnp.int32, (R, S), 1)
              ).astype(jnp.bfloat16)
    main = jnp.dot(x_ref[...], w1t_ref[...],
                   preferred_element_type=jnp.float32)
    gath = jnp.dot(onehot, g_ref[...], preferred_element_type=jnp.float32)
    out_ref[...] = main + gath


@jax.jit
def kernel(x, batch_id, W, b):
    ids = batch_id.astype(jnp.int32)
    wt = W.T
    w1t = wt[:D]
    w2t = wt[D:]
    brow = b.reshape(1, OUT)

    zsum = jnp.zeros((S, D), jnp.float32)
    zcnt = jnp.zeros((S, CW), jnp.float32)
    ones = jnp.ones((CH, CW), jnp.float32)

    sums, cnts = _sc_segsum(x, ids, zsum, zcnt, ones)

    return pl.pallas_call(
        _tc_body,
        grid=(NB,),
        in_specs=[
            pl.BlockSpec((R, D), lambda i: (i, 0)),
            pl.BlockSpec((R, 1), lambda i: (i, 0)),
            pl.BlockSpec((D, OUT), lambda i: (0, 0)),
            pl.BlockSpec((D, OUT), lambda i: (0, 0)),
            pl.BlockSpec((1, OUT), lambda i: (0, 0)),
            pl.BlockSpec((NC, S, D), lambda i: (0, 0, 0)),
            pl.BlockSpec((NC, S, CW), lambda i: (0, 0, 0)),
        ],
        out_specs=pl.BlockSpec((R, OUT), lambda i: (i, 0)),
        out_shape=jax.ShapeDtypeStruct((N, OUT), jnp.float32),
        scratch_shapes=[
            pltpu.VMEM((S, OUT), jnp.bfloat16),
        ],
    )(x, ids.reshape(N, 1), w1t, w2t, brow, sums, cnts)
